# Initial kernel scaffold; baseline (speedup 1.0000x reference)
import jax
import jax.numpy as jnp
from jax.experimental import pallas as pl


def kernel(x, g, r, s, r_ij, gcn2_w, gcn2_b, gcn3_w, gcn3_b, w10, w2_10, att10, w11, w2_11, att11):
    raise NotImplementedError("write your pallas kernel here")


# speedup vs baseline: 5.4814x; 5.4814x over previous
import functools

import jax
import jax.numpy as jnp
from jax import lax
from jax.experimental import pallas as pl
from jax.experimental.pallas import tpu as pltpu
from jax.experimental.pallas import tpu_sc as plsc

N = 10000
E = 320000
C = 128
NC = 2
NS = 16
NW = NC * NS
EW = E // NW
SUB = 80
ROWS2D = E // SUB

_f32 = jnp.float32
_i32 = jnp.int32


def _mesh():
    return plsc.VectorSubcoreMesh(
        core_axis_name="c", subcore_axis_name="s", num_cores=NC, num_subcores=NS
    )


def _wid():
    return lax.axis_index("c") * NS + lax.axis_index("s")


@functools.partial(
    pl.kernel,
    out_type=jax.ShapeDtypeStruct((NC * N,), _f32),
    mesh=_mesh(),
    scratch_types=[
        pltpu.VMEM((25, SUB), _i32),
        pltpu.VMEM((SUB,), _f32),
        pltpu.VMEM((2000,), _f32),
        pltpu.VMEM_SHARED((N,), _f32),
    ],
    name="sc_deg",
    compiler_params=pltpu.CompilerParams(needs_layout_passes=False),
)
def _sc_deg(row3a, ones_hbm, z2000, degp, idx_v, ones_v, zb_v, acc):
    cid = lax.axis_index("c")
    sid = lax.axis_index("s")
    wid = _wid()

    @pl.when(sid == 0)
    def _zero():
        pltpu.sync_copy(z2000, zb_v)
        for t in range(5):
            pltpu.sync_copy(zb_v, acc.at[pl.ds(t * 2000, 2000)])

    pltpu.sync_copy(ones_hbm, ones_v)
    plsc.subcore_barrier()

    def chunk(k, _):
        pltpu.sync_copy(row3a.at[wid * 5 + k], idx_v)
        for j in range(25):
            pltpu.sync_copy(ones_v, acc.at[idx_v.at[j]], add=True)
        return 0

    lax.fori_loop(0, 5, chunk, 0)
    plsc.subcore_barrier()

    @pl.when(sid == 0)
    def _out():
        for t in range(5):
            pltpu.sync_copy(acc.at[pl.ds(t * 2000, 2000)], zb_v)
            pltpu.sync_copy(zb_v, degp.at[pl.ds(cid * N + t * 2000, 2000)])


@functools.partial(
    pl.kernel,
    out_type=jax.ShapeDtypeStruct((NC, N, C), _f32),
    mesh=_mesh(),
    scratch_types=[
        pltpu.VMEM((5, SUB), _i32),
        pltpu.VMEM((5, SUB), _i32),
        pltpu.VMEM((SUB, C), _f32),
        pltpu.VMEM((SUB, C), _f32),
        pltpu.SemaphoreType.DMA,
        pltpu.SemaphoreType.DMA,
        pltpu.VMEM_SHARED((N, C), _f32),
    ],
    name="sc_gconv",
    compiler_params=pltpu.CompilerParams(needs_layout_passes=False),
)
def _sc_gconv(y, row3b, col3b, z80, part, idxr, idxc, buf_a, buf_b, semg, sems, acc):
    cid = lax.axis_index("c")
    sid = lax.axis_index("s")
    wid = _wid()
    bufs = (buf_a, buf_b)

    pltpu.sync_copy(z80, buf_a)
    for t in range(7):
        pltpu.sync_copy(buf_a, acc.at[pl.ds(sid * 624 + t * 80, 80), :])
    pltpu.sync_copy(
        buf_a.at[pl.ds(0, 64)], acc.at[pl.ds(sid * 624 + 560, 64), :]
    )

    @pl.when(sid == NS - 1)
    def _ztail():
        pltpu.sync_copy(buf_a.at[pl.ds(0, 16)], acc.at[pl.ds(9984, 16), :])

    plsc.subcore_barrier()

    def chunk(k, _):
        b = wid * 25 + k
        pltpu.sync_copy(row3b.at[b], idxr)
        pltpu.sync_copy(col3b.at[b], idxc)
        scat = []
        for j in range(5):
            buf = bufs[j % 2]
            g = pltpu.async_copy(y.at[idxr.at[j]], buf, semg)
            if scat:
                scat[-1].wait()
            g.wait()
            scat.append(
                pltpu.async_copy(buf, acc.at[idxc.at[j]], sems, add=True)
            )
        scat[-1].wait()
        return 0

    lax.fori_loop(0, 25, chunk, 0)
    plsc.subcore_barrier()
    for t in range(7):
        r0 = sid * 624 + t * 80
        pltpu.sync_copy(acc.at[pl.ds(r0, 80), :], buf_a)
        pltpu.sync_copy(buf_a, part.at[cid, pl.ds(r0, 80), :])
    r1 = sid * 624 + 560
    pltpu.sync_copy(acc.at[pl.ds(r1, 64), :], buf_a.at[pl.ds(0, 64)])
    pltpu.sync_copy(buf_a.at[pl.ds(0, 64)], part.at[cid, pl.ds(r1, 64), :])

    @pl.when(sid == NS - 1)
    def _ctail():
        pltpu.sync_copy(acc.at[pl.ds(9984, 16), :], buf_a.at[pl.ds(0, 16)])
        pltpu.sync_copy(buf_a.at[pl.ds(0, 16)], part.at[cid, pl.ds(9984, 16), :])


@functools.partial(
    pl.kernel,
    out_type=(
        jax.ShapeDtypeStruct((E,), _f32),
        jax.ShapeDtypeStruct((NC * N,), _f32),
    ),
    mesh=_mesh(),
    scratch_types=[
        pltpu.VMEM((N,), _f32),
        pltpu.VMEM((N,), _f32),
        pltpu.VMEM((2000,), _i32),
        pltpu.VMEM((2000,), _i32),
        pltpu.VMEM((25, SUB), _i32),
        pltpu.VMEM((2000,), _f32),
        pltpu.VMEM((2000,), _f32),
        pltpu.VMEM((16,), _f32),
        pltpu.VMEM_SHARED((N,), _f32),
    ],
    name="sc_softmax",
    compiler_params=pltpu.CompilerParams(needs_layout_passes=False),
)
def _sc_ab(ei, eja, ejb, row1d, col1d, col3a, kvec, z2000,
           ea_out, asump,
           ei_v, eja_v, rowf, colf, idxc2, ejb_v, ea_v, kv_v, acc):
    cid = lax.axis_index("c")
    sid = lax.axis_index("s")
    wid = _wid()

    @pl.when(sid == 0)
    def _zero():
        pltpu.sync_copy(z2000, ea_v)
        for t in range(5):
            pltpu.sync_copy(ea_v, acc.at[pl.ds(t * 2000, 2000)])

    pltpu.sync_copy(ei, ei_v)
    pltpu.sync_copy(eja, eja_v)
    pltpu.sync_copy(kvec, kv_v)
    plsc.subcore_barrier()
    kv = kv_v[...]

    def chunk(k, _):
        base = wid * EW + k * 2000
        pltpu.sync_copy(row1d.at[pl.ds(base, 2000)], rowf)
        pltpu.sync_copy(col1d.at[pl.ds(base, 2000)], colf)
        pltpu.sync_copy(col3a.at[wid * 5 + k], idxc2)
        pltpu.sync_copy(ejb.at[pl.ds(base, 2000)], ejb_v)
        for g in range(125):
            sl = pl.ds(g * 16, 16)
            a = (
                plsc.load_gather(ei_v, [colf[sl]])
                + plsc.load_gather(eja_v, [rowf[sl]])
                + ejb_v[sl]
            )
            a = jnp.maximum(a, 0.2 * a)
            ea_v[sl] = jnp.exp(a - kv)
        pltpu.sync_copy(ea_v, ea_out.at[pl.ds(base, 2000)])
        for j in range(25):
            pltpu.sync_copy(
                ea_v.at[pl.ds(j * SUB, SUB)], acc.at[idxc2.at[j]], add=True
            )
        return 0

    lax.fori_loop(0, 5, chunk, 0)
    plsc.subcore_barrier()

    @pl.when(sid == 0)
    def _out():
        for t in range(5):
            pltpu.sync_copy(acc.at[pl.ds(t * 2000, 2000)], ea_v)
            pltpu.sync_copy(ea_v, asump.at[pl.ds(cid * N + t * 2000, 2000)])


@functools.partial(
    pl.kernel,
    out_type=(
        jax.ShapeDtypeStruct((E, C), _f32),
        jax.ShapeDtypeStruct((E,), _f32),
    ),
    mesh=_mesh(),
    scratch_types=[
        pltpu.VMEM((1, SUB), _i32),
        pltpu.VMEM((1, SUB), _i32),
        pltpu.VMEM((SUB, C), _f32),
        pltpu.VMEM((SUB,), _f32),
        pltpu.SemaphoreType.DMA,
        pltpu.SemaphoreType.DMA,
    ],
    name="sc_gather",
    compiler_params=pltpu.CompilerParams(needs_layout_passes=False),
)
def _sc_gather(xa, winv, row3c, col3c, xarows, winvcol,
               idxr, idxc, buf, wbuf, semg, semw):
    wid = _wid()

    def chunk(k, _):
        base = wid * EW + k * SUB
        b = wid * 125 + k
        pltpu.sync_copy(row3c.at[b], idxr)
        pltpu.sync_copy(col3c.at[b], idxc)
        g = pltpu.async_copy(xa.at[idxr.at[0]], buf, semg)
        w = pltpu.async_copy(winv.at[idxc.at[0]], wbuf, semw)
        g.wait()
        pltpu.sync_copy(buf, xarows.at[pl.ds(base, SUB), :])
        w.wait()
        pltpu.sync_copy(wbuf, winvcol.at[pl.ds(base, SUB)])
        return 0

    lax.fori_loop(0, 125, chunk, 0)


@functools.partial(
    pl.kernel,
    out_type=jax.ShapeDtypeStruct((NC, N, C), _f32),
    mesh=_mesh(),
    scratch_types=[
        pltpu.VMEM((1, SUB), _i32),
        pltpu.VMEM((SUB, C), _f32),
        pltpu.SemaphoreType.DMA,
        pltpu.VMEM_SHARED((N, C), _f32),
    ],
    name="sc_scat",
    compiler_params=pltpu.CompilerParams(needs_layout_passes=False),
)
def _sc_scat(m, col3c, z80, part, idxc, buf_a, sems, acc):
    cid = lax.axis_index("c")
    sid = lax.axis_index("s")
    wid = _wid()

    pltpu.sync_copy(z80, buf_a)
    for t in range(7):
        pltpu.sync_copy(buf_a, acc.at[pl.ds(sid * 624 + t * 80, 80), :])
    pltpu.sync_copy(buf_a.at[pl.ds(0, 64)], acc.at[pl.ds(sid * 624 + 560, 64), :])

    @pl.when(sid == NS - 1)
    def _ztail():
        pltpu.sync_copy(buf_a.at[pl.ds(0, 16)], acc.at[pl.ds(9984, 16), :])

    plsc.subcore_barrier()

    def chunk(k, _):
        base = wid * EW + k * SUB
        b = wid * 125 + k
        pltpu.sync_copy(col3c.at[b], idxc)
        pltpu.sync_copy(m.at[pl.ds(base, SUB), :], buf_a)
        pltpu.sync_copy(buf_a, acc.at[idxc.at[0]], add=True)
        return 0

    lax.fori_loop(0, 125, chunk, 0)
    plsc.subcore_barrier()
    for t in range(7):
        r0 = sid * 624 + t * 80
        pltpu.sync_copy(acc.at[pl.ds(r0, 80), :], buf_a)
        pltpu.sync_copy(buf_a, part.at[cid, pl.ds(r0, 80), :])
    r1 = sid * 624 + 560
    pltpu.sync_copy(acc.at[pl.ds(r1, 64), :], buf_a.at[pl.ds(0, 64)])
    pltpu.sync_copy(buf_a.at[pl.ds(0, 64)], part.at[cid, pl.ds(r1, 64), :])

    @pl.when(sid == NS - 1)
    def _ctail():
        pltpu.sync_copy(acc.at[pl.ds(9984, 16), :], buf_a.at[pl.ds(0, 16)])
        pltpu.sync_copy(buf_a.at[pl.ds(0, 16)], part.at[cid, pl.ds(9984, 16), :])


_BE = 512


def _bigmm_body(rij_ref, s_ref, m10_ref, m11_ref, v10_ref, v11_ref,
                r10_ref, r11_ref, e10_ref, e11_ref, mx10_ref, mx11_ref):
    i = pl.program_id(0)
    rij = rij_ref[...] * s_ref[...]
    r10_ref[...] = jnp.dot(rij, m10_ref[...], preferred_element_type=_f32)
    r11_ref[...] = jnp.dot(rij, m11_ref[...], preferred_element_type=_f32)
    e10 = jnp.dot(rij, v10_ref[...], preferred_element_type=_f32)
    e11 = jnp.dot(rij, v11_ref[...], preferred_element_type=_f32)
    e10_ref[...] = e10
    e11_ref[...] = e11
    m10 = jnp.max(e10)
    m11 = jnp.max(e11)

    @pl.when(i == 0)
    def _init():
        mx10_ref[...] = jnp.full((1, 1), m10, _f32)
        mx11_ref[...] = jnp.full((1, 1), m11, _f32)

    @pl.when(i > 0)
    def _acc():
        mx10_ref[...] = jnp.maximum(mx10_ref[...], m10)
        mx11_ref[...] = jnp.maximum(mx11_ref[...], m11)


def _bigmm(r_ij, s2, m10, m11, v10, v11):
    nb = E // _BE
    return pl.pallas_call(
        _bigmm_body,
        grid=(nb,),
        in_specs=[
            pl.BlockSpec((_BE, C), lambda i: (i, 0)),
            pl.BlockSpec((_BE, 1), lambda i: (i, 0)),
            pl.BlockSpec((C, C), lambda i: (0, 0)),
            pl.BlockSpec((C, C), lambda i: (0, 0)),
            pl.BlockSpec((C, 1), lambda i: (0, 0)),
            pl.BlockSpec((C, 1), lambda i: (0, 0)),
        ],
        out_specs=[
            pl.BlockSpec((_BE, C), lambda i: (i, 0)),
            pl.BlockSpec((_BE, C), lambda i: (i, 0)),
            pl.BlockSpec((_BE, 1), lambda i: (i, 0)),
            pl.BlockSpec((_BE, 1), lambda i: (i, 0)),
            pl.BlockSpec((1, 1), lambda i: (0, 0)),
            pl.BlockSpec((1, 1), lambda i: (0, 0)),
        ],
        out_shape=[
            jax.ShapeDtypeStruct((E, C), _f32),
            jax.ShapeDtypeStruct((E, C), _f32),
            jax.ShapeDtypeStruct((E, 1), _f32),
            jax.ShapeDtypeStruct((E, 1), _f32),
            jax.ShapeDtypeStruct((1, 1), _f32),
            jax.ShapeDtypeStruct((1, 1), _f32),
        ],
    )(r_ij, s2, m10, m11, v10, v11)


_BN = 1000


def _node_body(from_parts, x_ref, w2a_ref, v1_ref, v2_ref,
               xa_ref, ei_ref, eja_ref, mei_ref, meja_ref):
    i = pl.program_id(0)
    if from_parts:
        xin = jax.nn.relu(x_ref[0] + x_ref[1])
    else:
        xin = x_ref[...]
    xa_ref[...] = jnp.dot(xin, w2a_ref[...], preferred_element_type=_f32)
    e1 = jnp.dot(xin, v1_ref[...], preferred_element_type=_f32)
    e2 = jnp.dot(xin, v2_ref[...], preferred_element_type=_f32)
    ei_ref[...] = e1
    eja_ref[...] = e2
    m1 = jnp.max(e1)
    m2 = jnp.max(e2)

    @pl.when(i == 0)
    def _init():
        mei_ref[...] = jnp.full((1, 1), m1, _f32)
        meja_ref[...] = jnp.full((1, 1), m2, _f32)

    @pl.when(i > 0)
    def _acc():
        mei_ref[...] = jnp.maximum(mei_ref[...], m1)
        meja_ref[...] = jnp.maximum(meja_ref[...], m2)


def _node(xin, w2a, v1, v2, from_parts):
    nb = N // _BN
    xspec = (
        pl.BlockSpec((NC, _BN, C), lambda i: (0, i, 0))
        if from_parts
        else pl.BlockSpec((_BN, C), lambda i: (i, 0))
    )
    return pl.pallas_call(
        functools.partial(_node_body, from_parts),
        grid=(nb,),
        in_specs=[
            xspec,
            pl.BlockSpec((C, C), lambda i: (0, 0)),
            pl.BlockSpec((C, 1), lambda i: (0, 0)),
            pl.BlockSpec((C, 1), lambda i: (0, 0)),
        ],
        out_specs=[
            pl.BlockSpec((_BN, C), lambda i: (i, 0)),
            pl.BlockSpec((_BN, 1), lambda i: (i, 0)),
            pl.BlockSpec((_BN, 1), lambda i: (i, 0)),
            pl.BlockSpec((1, 1), lambda i: (0, 0)),
            pl.BlockSpec((1, 1), lambda i: (0, 0)),
        ],
        out_shape=[
            jax.ShapeDtypeStruct((N, C), _f32),
            jax.ShapeDtypeStruct((N, 1), _f32),
            jax.ShapeDtypeStruct((N, 1), _f32),
            jax.ShapeDtypeStruct((1, 1), _f32),
            jax.ShapeDtypeStruct((1, 1), _f32),
        ],
    )(xin, w2a, v1, v2)


def _gcnprep_body(dp_ref, x_ref, w_ref, dinv_ref, y_ref):
    d = dp_ref[0] + dp_ref[1] + 1.0
    dv = lax.rsqrt(d)
    dinv_ref[...] = dv
    y_ref[...] = dv * (x_ref[...] * w_ref[...])


def _gcnprep(degp3, x, w):
    nb = N // _BN
    return pl.pallas_call(
        _gcnprep_body,
        grid=(nb,),
        in_specs=[
            pl.BlockSpec((NC, _BN, 1), lambda i: (0, i, 0)),
            pl.BlockSpec((_BN, C), lambda i: (i, 0)),
            pl.BlockSpec((1, C), lambda i: (0, 0)),
        ],
        out_specs=[
            pl.BlockSpec((_BN, 1), lambda i: (i, 0)),
            pl.BlockSpec((_BN, C), lambda i: (i, 0)),
        ],
        out_shape=[
            jax.ShapeDtypeStruct((N, 1), _f32),
            jax.ShapeDtypeStruct((N, C), _f32),
        ],
    )(degp3, x, w)


def _gcnmid_body(p_ref, y_ref, dinv_ref, b_ref, w3_ref, y2_ref):
    dv = dinv_ref[...]
    h = dv * (p_ref[0] + p_ref[1] + y_ref[...]) + b_ref[...]
    h = jax.nn.relu(h)
    y2_ref[...] = dv * (h * w3_ref[...])


def _gcnmid(part, y1, dinv, b2, w3):
    nb = N // _BN
    return pl.pallas_call(
        _gcnmid_body,
        grid=(nb,),
        in_specs=[
            pl.BlockSpec((NC, _BN, C), lambda i: (0, i, 0)),
            pl.BlockSpec((_BN, C), lambda i: (i, 0)),
            pl.BlockSpec((_BN, 1), lambda i: (i, 0)),
            pl.BlockSpec((1, C), lambda i: (0, 0)),
            pl.BlockSpec((1, C), lambda i: (0, 0)),
        ],
        out_specs=[pl.BlockSpec((_BN, C), lambda i: (i, 0))],
        out_shape=[jax.ShapeDtypeStruct((N, C), _f32)],
    )(part, y1, dinv, b2, w3)[0]


def _winv_body(a_ref, w_ref):
    w_ref[...] = 1.0 / (a_ref[0:8] + a_ref[8:16] + 1e-16)


def _winv(asump):
    out = pl.pallas_call(
        _winv_body,
        out_shape=jax.ShapeDtypeStruct((8, N // 8), _f32),
    )(asump.reshape(16, N // 8))
    return out.reshape(N)


def _relmsg_body(xr_ref, rb_ref, ea_ref, wc_ref, m_ref):
    w = ea_ref[...] * wc_ref[...]
    m_ref[...] = (xr_ref[...] + rb_ref[...]) * w


def _relmsg(xarows, rbp, ea2, wc2):
    nb = E // _BE
    return pl.pallas_call(
        _relmsg_body,
        grid=(nb,),
        in_specs=[
            pl.BlockSpec((_BE, C), lambda i: (i, 0)),
            pl.BlockSpec((_BE, C), lambda i: (i, 0)),
            pl.BlockSpec((_BE, 1), lambda i: (i, 0)),
            pl.BlockSpec((_BE, 1), lambda i: (i, 0)),
        ],
        out_specs=[pl.BlockSpec((_BE, C), lambda i: (i, 0))],
        out_shape=[jax.ShapeDtypeStruct((E, C), _f32)],
    )(xarows, rbp, ea2, wc2)[0]


def _l2n(t):
    n = jnp.sqrt(jnp.sum(t * t, axis=1, keepdims=True))
    return t / jnp.maximum(n, 1e-12)


def _final_body(p_ref, y2_ref, dinv_ref, b3_ref, macc_ref, o1_ref, o2_ref):
    dv = dinv_ref[...]
    t = dv * (p_ref[0] + p_ref[1] + y2_ref[...]) + b3_ref[...]
    o1_ref[...] = _l2n(t)
    o2_ref[...] = _l2n(macc_ref[0] + macc_ref[1])


def _final(part2, y2, dinv, b3, maccb):
    nb = N // _BN
    return pl.pallas_call(
        _final_body,
        grid=(nb,),
        in_specs=[
            pl.BlockSpec((NC, _BN, C), lambda i: (0, i, 0)),
            pl.BlockSpec((_BN, C), lambda i: (i, 0)),
            pl.BlockSpec((_BN, 1), lambda i: (i, 0)),
            pl.BlockSpec((1, C), lambda i: (0, 0)),
            pl.BlockSpec((NC, _BN, C), lambda i: (0, i, 0)),
        ],
        out_specs=[
            pl.BlockSpec((_BN, C), lambda i: (i, 0)),
            pl.BlockSpec((_BN, C), lambda i: (i, 0)),
        ],
        out_shape=[
            jax.ShapeDtypeStruct((N, C), _f32),
            jax.ShapeDtypeStruct((N, C), _f32),
        ],
    )(part2, y2, dinv, b3, maccb)


def kernel(x, g, r, s, r_ij, gcn2_w, gcn2_b, gcn3_w, gcn3_b,
           w10, w2_10, att10, w11, w2_11, att11):
    del r
    row = g[0]
    col = g[1]
    row3a = row.reshape(NW * 5, 25, SUB)
    col3a = col.reshape(NW * 5, 25, SUB)
    row3b = row.reshape(NW * 25, 5, SUB)
    col3b = col.reshape(NW * 25, 5, SUB)
    row3c = row.reshape(NW * 125, 1, SUB)
    col3c = col.reshape(NW * 125, 1, SUB)
    ones80 = jnp.ones((SUB,), _f32)
    z2000 = jnp.zeros((2000,), _f32)
    z80 = jnp.zeros((SUB, C), _f32)

    degp = _sc_deg(row3a, ones80, z2000)
    dinv, y1 = _gcnprep(degp.reshape(NC, N, 1), x, gcn2_w)
    p1 = _sc_gconv(y1, row3b, col3b, z80)
    y2 = _gcnmid(p1, y1, dinv, gcn2_b.reshape(1, C), gcn3_w)
    p2 = _sc_gconv(y2, row3b, col3b, z80)

    def combos(W, W2, att):
        a = att.reshape(-1)
        att_i, att_j = a[:C], a[C:]
        w2a, w2b = W2[:C], W2[C:]
        v1 = (W @ att_i)[:, None]
        v2 = (w2a @ att_j)[:, None]
        v3 = (w2b @ att_j)[:, None]
        m = w2b + jnp.eye(C, dtype=_f32)
        return w2a, v1, v2, v3, m

    w2a10, v1_10, v2_10, v3_10, m10 = combos(w10, w2_10, att10)
    w2a11, v1_11, v2_11, v3_11, m11 = combos(w11, w2_11, att11)

    rbp10, rbp11, ejb10, ejb11, mx10, mx11 = _bigmm(
        r_ij, s.reshape(E, 1), m10, m11, v3_10, v3_11
    )

    xa10, ei10, eja10, me10, mea10 = _node(x, w2a10, v1_10, v2_10, False)
    k10 = jnp.full((16,), me10[0, 0] + mea10[0, 0] + mx10[0, 0], _f32)
    ea10, asum10 = _sc_ab(
        ei10.reshape(N), eja10.reshape(N), ejb10.reshape(E),
        row, col, col3a, k10, z2000,
    )
    xr10, wc10 = _sc_gather(xa10, _winv(asum10), row3c, col3c)
    m10e = _relmsg(xr10, rbp10, ea10.reshape(E, 1), wc10.reshape(E, 1))
    macc10 = _sc_scat(m10e, col3c, z80)

    xa11, ei11, eja11, me11, mea11 = _node(macc10, w2a11, v1_11, v2_11, True)
    k11 = jnp.full((16,), me11[0, 0] + mea11[0, 0] + mx11[0, 0], _f32)
    ea11, asum11 = _sc_ab(
        ei11.reshape(N), eja11.reshape(N), ejb11.reshape(E),
        row, col, col3a, k11, z2000,
    )
    xr11, wc11 = _sc_gather(xa11, _winv(asum11), row3c, col3c)
    m11e = _relmsg(xr11, rbp11, ea11.reshape(E, 1), wc11.reshape(E, 1))
    macc11 = _sc_scat(m11e, col3c, z80)

    o1, o2 = _final(p2, y2, dinv, gcn3_b.reshape(1, C), macc11)
    return (o1, o2)
